# Initial kernel scaffold; baseline (speedup 1.0000x reference)
import jax
import jax.numpy as jnp
from jax.experimental import pallas as pl


def kernel(hidden_states, W):
    raise NotImplementedError("write your pallas kernel here")


# speedup vs baseline: 1.5575x; 1.5575x over previous
import functools

import jax
import jax.numpy as jnp
from jax.experimental import pallas as pl
from jax.experimental.pallas import tpu as pltpu

_B, _S, _D, _E, _K = 4, 2048, 2048, 64, 2
_AUX_LOSS_COEF = 0.01
_TOK_BLK = 512


def _router_body(x_ref, w_ref, probs_ref, idx_ref, loss_ref, cnt_acc, sp_acc):
    i = pl.program_id(0)
    nsteps = pl.num_programs(0)

    x = x_ref[...]
    w = w_ref[...]
    logits = jax.lax.dot_general(
        x, w, (((1,), (1,)), ((), ())), preferred_element_type=jnp.float32
    )

    m = jnp.max(logits, axis=-1, keepdims=True)
    e = jnp.exp(logits - m)
    s = jnp.sum(e, axis=-1, keepdims=True)
    probs = e / s

    iota = jax.lax.broadcasted_iota(jnp.int32, probs.shape, 1)
    p1 = jnp.max(probs, axis=-1, keepdims=True)
    i1 = jnp.min(jnp.where(probs == p1, iota, _E), axis=-1, keepdims=True)
    masked = jnp.where(iota == i1, -jnp.inf, probs)
    p2 = jnp.max(masked, axis=-1, keepdims=True)
    i2 = jnp.min(jnp.where(masked == p2, iota, _E), axis=-1, keepdims=True)

    denom = p1 + p2
    probs_ref[...] = jnp.concatenate([p1 / denom, p2 / denom], axis=-1)
    idx_ref[...] = jnp.concatenate([i1, i2], axis=-1)

    blk_cnt = (
        (iota == i1).astype(jnp.float32) + (iota == i2).astype(jnp.float32)
    ).sum(axis=0, keepdims=True)
    blk_sp = probs.sum(axis=0, keepdims=True)

    @pl.when(i == 0)
    def _init():
        cnt_acc[...] = blk_cnt
        sp_acc[...] = blk_sp

    @pl.when(i > 0)
    def _acc():
        cnt_acc[...] += blk_cnt
        sp_acc[...] += blk_sp

    @pl.when(i == nsteps - 1)
    def _fin():
        n_tok = jnp.float32(_B * _S)
        aux = _E * jnp.sum(cnt_acc[...] * sp_acc[...], keepdims=True) / (n_tok * n_tok)
        loss_ref[...] = aux.reshape(1, 1) * _AUX_LOSS_COEF


@functools.partial(jax.jit, static_argnames=())
def kernel(hidden_states, W):
    n_tok = _B * _S
    x = hidden_states.reshape(n_tok, _D)
    nsteps = n_tok // _TOK_BLK

    probs, idx, loss = pl.pallas_call(
        _router_body,
        grid=(nsteps,),
        in_specs=[
            pl.BlockSpec((_TOK_BLK, _D), lambda i: (i, 0)),
            pl.BlockSpec((_E, _D), lambda i: (0, 0)),
        ],
        out_specs=[
            pl.BlockSpec((_TOK_BLK, _K), lambda i: (i, 0)),
            pl.BlockSpec((_TOK_BLK, _K), lambda i: (i, 0)),
            pl.BlockSpec((1, 1), lambda i: (0, 0)),
        ],
        out_shape=[
            jax.ShapeDtypeStruct((n_tok, _K), jnp.float32),
            jax.ShapeDtypeStruct((n_tok, _K), jnp.int32),
            jax.ShapeDtypeStruct((1, 1), jnp.float32),
        ],
        scratch_shapes=[
            pltpu.VMEM((1, _E), jnp.float32),
            pltpu.VMEM((1, _E), jnp.float32),
        ],
    )(x, W)

    return (
        probs.reshape(_B, _S, _K),
        idx.reshape(_B, _S, _K),
        loss.reshape(()),
    )


# speedup vs baseline: 1.5843x; 1.0172x over previous
import functools

import jax
import jax.numpy as jnp
from jax.experimental import pallas as pl
from jax.experimental.pallas import tpu as pltpu

_B, _S, _D, _E, _K = 4, 2048, 2048, 64, 2
_AUX_LOSS_COEF = 0.01
_TOK_BLK = 512


def _router_body(x_ref, w_ref, probs_ref, idx_ref, loss_ref, cnt_acc, sp_acc):
    i = pl.program_id(0)
    nsteps = pl.num_programs(0)

    x = x_ref[...]
    w = w_ref[...]
    logits = jax.lax.dot_general(
        x, w, (((1,), (1,)), ((), ())), preferred_element_type=jnp.float32
    )

    iota = jax.lax.broadcasted_iota(jnp.int32, logits.shape, 1)
    m1 = jnp.max(logits, axis=-1, keepdims=True)
    i1 = jnp.min(jnp.where(logits == m1, iota, _E), axis=-1, keepdims=True)
    masked = jnp.where(iota == i1, -jnp.inf, logits)
    m2 = jnp.max(masked, axis=-1, keepdims=True)
    i2 = jnp.min(jnp.where(masked == m2, iota, _E), axis=-1, keepdims=True)

    p1 = 1.0 / (1.0 + jnp.exp(m2 - m1))
    probs_ref[...] = jnp.concatenate([p1, 1.0 - p1], axis=-1)
    idx_ref[...] = jnp.concatenate([i1, i2], axis=-1)

    e = jnp.exp(logits - m1)
    r = 1.0 / jnp.sum(e, axis=-1, keepdims=True)

    blk_cnt = (
        (iota == i1).astype(jnp.float32) + (iota == i2).astype(jnp.float32)
    ).sum(axis=0, keepdims=True)
    blk_sp = (e * r).sum(axis=0, keepdims=True)

    @pl.when(i == 0)
    def _init():
        cnt_acc[...] = blk_cnt
        sp_acc[...] = blk_sp

    @pl.when(i > 0)
    def _acc():
        cnt_acc[...] += blk_cnt
        sp_acc[...] += blk_sp

    @pl.when(i == nsteps - 1)
    def _fin():
        n_tok = jnp.float32(_B * _S)
        aux = _E * jnp.sum(cnt_acc[...] * sp_acc[...], keepdims=True) / (n_tok * n_tok)
        loss_ref[...] = aux.reshape(1, 1) * _AUX_LOSS_COEF


@functools.partial(jax.jit, static_argnames=())
def kernel(hidden_states, W):
    n_tok = _B * _S
    x = hidden_states.reshape(n_tok, _D)
    nsteps = n_tok // _TOK_BLK

    probs, idx, loss = pl.pallas_call(
        _router_body,
        grid=(nsteps,),
        in_specs=[
            pl.BlockSpec((_TOK_BLK, _D), lambda i: (i, 0)),
            pl.BlockSpec((_E, _D), lambda i: (0, 0)),
        ],
        out_specs=[
            pl.BlockSpec((_TOK_BLK, _K), lambda i: (i, 0)),
            pl.BlockSpec((_TOK_BLK, _K), lambda i: (i, 0)),
            pl.BlockSpec((1, 1), lambda i: (0, 0)),
        ],
        out_shape=[
            jax.ShapeDtypeStruct((n_tok, _K), jnp.float32),
            jax.ShapeDtypeStruct((n_tok, _K), jnp.int32),
            jax.ShapeDtypeStruct((1, 1), jnp.float32),
        ],
        scratch_shapes=[
            pltpu.VMEM((1, _E), jnp.float32),
            pltpu.VMEM((1, _E), jnp.float32),
        ],
    )(x, W)

    return (
        probs.reshape(_B, _S, _K),
        idx.reshape(_B, _S, _K),
        loss.reshape(()),
    )


# speedup vs baseline: 1.8114x; 1.1433x over previous
import functools

import jax
import jax.numpy as jnp
from jax.experimental import pallas as pl
from jax.experimental.pallas import tpu as pltpu

_B, _S, _D, _E, _K = 4, 2048, 2048, 64, 2
_AUX_LOSS_COEF = 0.01
_TOK_BLK = 1024


def _router_body(x_ref, w_ref, probs_ref, idx_ref, loss_ref, cnt_acc, sp_acc):
    i = pl.program_id(0)
    nsteps = pl.num_programs(0)

    x = x_ref[...]
    w = w_ref[...]
    logits = jax.lax.dot_general(
        x, w, (((1,), (1,)), ((), ())), preferred_element_type=jnp.float32
    )

    iota = jax.lax.broadcasted_iota(jnp.int32, logits.shape, 1)
    m1 = jnp.max(logits, axis=-1, keepdims=True)
    i1 = jnp.min(jnp.where(logits == m1, iota, _E), axis=-1, keepdims=True)
    masked = jnp.where(iota == i1, -jnp.inf, logits)
    m2 = jnp.max(masked, axis=-1, keepdims=True)
    i2 = jnp.min(jnp.where(masked == m2, iota, _E), axis=-1, keepdims=True)

    p1 = 1.0 / (1.0 + jnp.exp(m2 - m1))
    probs_ref[...] = jnp.concatenate([p1, 1.0 - p1], axis=-1)
    idx_ref[...] = jnp.concatenate([i1, i2], axis=-1)

    e = jnp.exp(logits - m1)
    r = 1.0 / jnp.sum(e, axis=-1, keepdims=True)

    blk_cnt = (
        (iota == i1).astype(jnp.float32) + (iota == i2).astype(jnp.float32)
    ).sum(axis=0, keepdims=True)
    blk_sp = (e * r).sum(axis=0, keepdims=True)

    @pl.when(i == 0)
    def _init():
        cnt_acc[...] = blk_cnt
        sp_acc[...] = blk_sp

    @pl.when(i > 0)
    def _acc():
        cnt_acc[...] += blk_cnt
        sp_acc[...] += blk_sp

    @pl.when(i == nsteps - 1)
    def _fin():
        n_tok = jnp.float32(_B * _S)
        aux = _E * jnp.sum(cnt_acc[...] * sp_acc[...], keepdims=True) / (n_tok * n_tok)
        loss_ref[...] = aux.reshape(1, 1) * _AUX_LOSS_COEF


@functools.partial(jax.jit, static_argnames=())
def kernel(hidden_states, W):
    n_tok = _B * _S
    x = hidden_states.reshape(n_tok, _D)
    nsteps = n_tok // _TOK_BLK

    probs, idx, loss = pl.pallas_call(
        _router_body,
        grid=(nsteps,),
        in_specs=[
            pl.BlockSpec((_TOK_BLK, _D), lambda i: (i, 0)),
            pl.BlockSpec((_E, _D), lambda i: (0, 0)),
        ],
        out_specs=[
            pl.BlockSpec((_TOK_BLK, _K), lambda i: (i, 0)),
            pl.BlockSpec((_TOK_BLK, _K), lambda i: (i, 0)),
            pl.BlockSpec((1, 1), lambda i: (0, 0)),
        ],
        out_shape=[
            jax.ShapeDtypeStruct((n_tok, _K), jnp.float32),
            jax.ShapeDtypeStruct((n_tok, _K), jnp.int32),
            jax.ShapeDtypeStruct((1, 1), jnp.float32),
        ],
        scratch_shapes=[
            pltpu.VMEM((1, _E), jnp.float32),
            pltpu.VMEM((1, _E), jnp.float32),
        ],
    )(x, W)

    return (
        probs.reshape(_B, _S, _K),
        idx.reshape(_B, _S, _K),
        loss.reshape(()),
    )


# speedup vs baseline: 1.8196x; 1.0045x over previous
import functools

import jax
import jax.numpy as jnp
from jax.experimental import pallas as pl
from jax.experimental.pallas import tpu as pltpu

_B, _S, _D, _E, _K = 4, 2048, 2048, 64, 2
_AUX_LOSS_COEF = 0.01
_TOK_BLK = 2048


def _router_body(x_ref, w_ref, probs_ref, idx_ref, loss_ref, cnt_acc, sp_acc):
    i = pl.program_id(0)
    nsteps = pl.num_programs(0)

    x = x_ref[...]
    w = w_ref[...]
    logits = jax.lax.dot_general(
        x, w, (((1,), (1,)), ((), ())), preferred_element_type=jnp.float32
    )

    iota = jax.lax.broadcasted_iota(jnp.int32, logits.shape, 1)
    m1 = jnp.max(logits, axis=-1, keepdims=True)
    i1 = jnp.min(jnp.where(logits == m1, iota, _E), axis=-1, keepdims=True)
    masked = jnp.where(iota == i1, -jnp.inf, logits)
    m2 = jnp.max(masked, axis=-1, keepdims=True)
    i2 = jnp.min(jnp.where(masked == m2, iota, _E), axis=-1, keepdims=True)

    p1 = 1.0 / (1.0 + jnp.exp(m2 - m1))
    probs_ref[...] = jnp.concatenate([p1, 1.0 - p1], axis=-1)
    idx_ref[...] = jnp.concatenate([i1, i2], axis=-1)

    e = jnp.exp(logits - m1)
    r = 1.0 / jnp.sum(e, axis=-1, keepdims=True)

    blk_cnt = (
        (iota == i1).astype(jnp.float32) + (iota == i2).astype(jnp.float32)
    ).sum(axis=0, keepdims=True)
    blk_sp = (e * r).sum(axis=0, keepdims=True)

    @pl.when(i == 0)
    def _init():
        cnt_acc[...] = blk_cnt
        sp_acc[...] = blk_sp

    @pl.when(i > 0)
    def _acc():
        cnt_acc[...] += blk_cnt
        sp_acc[...] += blk_sp

    @pl.when(i == nsteps - 1)
    def _fin():
        n_tok = jnp.float32(_B * _S)
        aux = _E * jnp.sum(cnt_acc[...] * sp_acc[...], keepdims=True) / (n_tok * n_tok)
        loss_ref[...] = aux.reshape(1, 1) * _AUX_LOSS_COEF


@functools.partial(jax.jit, static_argnames=())
def kernel(hidden_states, W):
    n_tok = _B * _S
    x = hidden_states.reshape(n_tok, _D)
    nsteps = n_tok // _TOK_BLK

    probs, idx, loss = pl.pallas_call(
        _router_body,
        grid=(nsteps,),
        in_specs=[
            pl.BlockSpec((_TOK_BLK, _D), lambda i: (i, 0)),
            pl.BlockSpec((_E, _D), lambda i: (0, 0)),
        ],
        out_specs=[
            pl.BlockSpec((_TOK_BLK, _K), lambda i: (i, 0)),
            pl.BlockSpec((_TOK_BLK, _K), lambda i: (i, 0)),
            pl.BlockSpec((1, 1), lambda i: (0, 0)),
        ],
        out_shape=[
            jax.ShapeDtypeStruct((n_tok, _K), jnp.float32),
            jax.ShapeDtypeStruct((n_tok, _K), jnp.int32),
            jax.ShapeDtypeStruct((1, 1), jnp.float32),
        ],
        scratch_shapes=[
            pltpu.VMEM((1, _E), jnp.float32),
            pltpu.VMEM((1, _E), jnp.float32),
        ],
    )(x, W)

    return (
        probs.reshape(_B, _S, _K),
        idx.reshape(_B, _S, _K),
        loss.reshape(()),
    )
